# baseline (speedup 1.0000x reference)
import jax
import jax.numpy as jnp
from jax import lax
from jax.experimental import pallas as pl
from jax.experimental.pallas import tpu as pltpu
from jax.experimental.pallas import tpu_sc as plsc

D = 64
BW = 128
LCH = 2
LBLK = 8
NW = 32
VPAD = 512


def _pe_body(emb_hbm, dates_hbm, table_hbm, out_hbm,
             table_v, idx_v, emb_v, semt, semi, seme, semo):
    wid = lax.axis_index("s") * 2 + lax.axis_index("c")
    L = emb_hbm.shape[0]
    wb = pl.multiple_of(wid * BW, BW)
    nblocks = L // LBLK
    nchunks = L // LCH
    CPB = LBLK // LCH

    def idx_copy(blk):
        lbase = pl.multiple_of(blk * LBLK, LBLK)
        return pltpu.make_async_copy(
            dates_hbm.at[pl.ds(lbase, LBLK), pl.ds(wb, BW)],
            idx_v.at[lax.rem(blk, 2)], semi)

    def emb_copy(k):
        lbase = pl.multiple_of(k * LCH, LCH)
        return pltpu.make_async_copy(
            emb_hbm.at[pl.ds(lbase, LCH), :, pl.ds(wb, BW)],
            emb_v.at[lax.rem(k, 4)], seme)

    def out_copy(k):
        lbase = pl.multiple_of(k * LCH, LCH)
        return pltpu.make_async_copy(
            emb_v.at[lax.rem(k, 4)],
            out_hbm.at[pl.ds(lbase, LCH), :, pl.ds(wb, BW)], semo)

    tcopies = [
        pltpu.make_async_copy(
            table_hbm.at[dd], table_v.at[pl.ds(dd * VPAD, VPAD)], semt)
        for dd in range(D)
    ]
    for cp in tcopies:
        cp.start()
    idx_copy(0).start()
    idx_copy(1).start()
    emb_copy(0).start()
    emb_copy(1).start()
    for cp in tcopies:
        cp.wait()
    idx_copy(0).wait()

    def chunk_body(k, carry):
        s = lax.rem(k, 4)
        blk = k // CPB
        sb = lax.rem(blk, 2)

        @pl.when(k >= 2)
        def _():
            out_copy(k - 2).wait()

        @pl.when(k + 2 < nchunks)
        def _():
            emb_copy(k + 2).start()

        @pl.when((k + 1 < nchunks) & (lax.rem(k + 1, CPB) == 0))
        def _():
            idx_copy(blk + 1).wait()

        emb_copy(k).wait()

        NG = BW // 16
        sls = [pl.ds(g * 16, 16) for g in range(NG)]
        for li in range(LCH):
            lrow = lax.rem(k, CPB) * LCH + li
            idx16s = [idx_v[sb, lrow, sls[g]] for g in range(NG)]
            prev_vs = None
            for dd in range(D):
                vs = [plsc.load_gather(table_v, [idx16s[g] + dd * VPAD])
                      for g in range(NG)]
                if prev_vs is not None:
                    for g in range(NG):
                        plsc.addupdate(
                            emb_v.at[s, li, dd - 1, sls[g]], prev_vs[g])
                prev_vs = vs
            for g in range(NG):
                plsc.addupdate(emb_v.at[s, li, D - 1, sls[g]], prev_vs[g])

        @pl.when((lax.rem(k, CPB) == CPB - 1) & (blk + 2 < nblocks))
        def _():
            idx_copy(blk + 2).start()

        out_copy(k).start()
        return carry

    lax.fori_loop(0, nchunks, chunk_body, 0)
    out_copy(nchunks - 2).wait()
    out_copy(nchunks - 1).wait()


def kernel(emb, dates, emb_table):
    B, L, d = emb.shape
    emb_t = jnp.transpose(emb, (1, 2, 0))
    dates_t = dates.T
    table_t = jnp.pad(emb_table.T, ((0, 0), (0, VPAD - emb_table.shape[0])))

    mesh = plsc.VectorSubcoreMesh(core_axis_name="c", subcore_axis_name="s")
    pe = pl.kernel(
        _pe_body,
        out_type=jax.ShapeDtypeStruct((L, d, B), jnp.float32),
        mesh=mesh,
        compiler_params=pltpu.CompilerParams(needs_layout_passes=False),
        scratch_types=[
            pltpu.VMEM((D * VPAD,), jnp.float32),
            pltpu.VMEM((2, LBLK, BW), jnp.int32),
            pltpu.VMEM((4, LCH, D, BW), jnp.float32),
            pltpu.SemaphoreType.DMA,
            pltpu.SemaphoreType.DMA,
            pltpu.SemaphoreType.DMA,
            pltpu.SemaphoreType.DMA,
        ],
    )
    out_t = pe(emb_t, dates_t, table_t)
    return jnp.transpose(out_t, (2, 0, 1))


# speedup vs baseline: 1.9548x; 1.9548x over previous
import jax
import jax.numpy as jnp
from jax import lax
from jax.experimental import pallas as pl
from jax.experimental.pallas import tpu as pltpu
from jax.experimental.pallas import tpu_sc as plsc

D = 64
BW = 128
LBLK = 8
NW = 32
VPAD = 512


def _pe_body(emb_hbm, dates_hbm, table_hbm, out_hbm,
             table_v, idx_v, emb_v, semt, semi, seme, semo):
    wid = lax.axis_index("s") * 2 + lax.axis_index("c")
    L = emb_hbm.shape[0]
    wb = pl.multiple_of(wid * BW, BW)
    nblocks = L // LBLK

    def idx_copy(blk):
        lbase = pl.multiple_of(blk * LBLK, LBLK)
        return pltpu.make_async_copy(
            dates_hbm.at[pl.ds(lbase, LBLK), pl.ds(wb, BW)],
            idx_v.at[lax.rem(blk, 2)], semi)

    def emb_copy(k):
        return pltpu.make_async_copy(
            emb_hbm.at[k, :, pl.ds(wb, BW)],
            emb_v.at[lax.rem(k, 8)], seme)

    def out_copy(k):
        return pltpu.make_async_copy(
            emb_v.at[lax.rem(k, 8)],
            out_hbm.at[k, :, pl.ds(wb, BW)], semo)

    tcopies = [
        pltpu.make_async_copy(
            table_hbm.at[dd], table_v.at[pl.ds(dd * VPAD, VPAD)], semt)
        for dd in range(D)
    ]
    for cp in tcopies:
        cp.start()
    idx_copy(0).start()
    idx_copy(1).start()
    emb_copy(0).start()
    emb_copy(1).start()
    emb_copy(2).start()
    emb_copy(3).start()
    for cp in tcopies:
        cp.wait()
    idx_copy(0).wait()

    def chunk_body(k, carry):
        s = lax.rem(k, 8)
        blk = k // LBLK
        lrow = lax.rem(k, LBLK)

        @pl.when(k >= 4)
        def _():
            out_copy(k - 4).wait()

        @pl.when(k + 4 < L)
        def _():
            emb_copy(k + 4).start()

        @pl.when((k + 1 < L) & (lax.rem(k + 1, LBLK) == 0))
        def _():
            idx_copy(blk + 1).wait()

        emb_copy(k).wait()

        NG = BW // 16
        sls = [pl.ds(g * 16, 16) for g in range(NG)]
        idx16s = [idx_v[lax.rem(blk, 2), lrow, sls[g]] for g in range(NG)]
        prev_vs = None
        for dd in range(D):
            vs = [plsc.load_gather(table_v, [idx16s[g] + dd * VPAD])
                  for g in range(NG)]
            if prev_vs is not None:
                for g in range(NG):
                    plsc.addupdate(emb_v.at[s, dd - 1, sls[g]], prev_vs[g])
            prev_vs = vs
        for g in range(NG):
            plsc.addupdate(emb_v.at[s, D - 1, sls[g]], prev_vs[g])

        @pl.when((lrow == LBLK - 1) & (blk + 2 < nblocks))
        def _():
            idx_copy(blk + 2).start()

        out_copy(k).start()
        return carry

    lax.fori_loop(0, L, chunk_body, 0)
    out_copy(L - 4).wait()
    out_copy(L - 3).wait()
    out_copy(L - 2).wait()
    out_copy(L - 1).wait()


def kernel(emb, dates, emb_table):
    B, L, d = emb.shape
    emb_t = jnp.transpose(emb, (1, 2, 0))
    dates_t = dates.T
    table_t = jnp.pad(emb_table.T, ((0, 0), (0, VPAD - emb_table.shape[0])))

    mesh = plsc.VectorSubcoreMesh(core_axis_name="c", subcore_axis_name="s")
    pe = pl.kernel(
        _pe_body,
        out_type=jax.ShapeDtypeStruct((L, d, B), jnp.float32),
        mesh=mesh,
        compiler_params=pltpu.CompilerParams(needs_layout_passes=False),
        scratch_types=[
            pltpu.VMEM((D * VPAD,), jnp.float32),
            pltpu.VMEM((2, LBLK, BW), jnp.int32),
            pltpu.VMEM((8, D, BW), jnp.float32),
            pltpu.SemaphoreType.DMA,
            pltpu.SemaphoreType.DMA,
            pltpu.SemaphoreType.DMA,
            pltpu.SemaphoreType.DMA,
        ],
    )
    out_t = pe(emb_t, dates_t, table_t)
    return jnp.transpose(out_t, (2, 0, 1))
